# Initial kernel scaffold; baseline (speedup 1.0000x reference)
import jax
import jax.numpy as jnp
from jax.experimental import pallas as pl


def kernel(x_norm, in_proj_w, conv_w, conv_b, x_proj_w, dt_proj_w, dt_proj_b, out_proj_w, A_log, D):
    raise NotImplementedError("write your pallas kernel here")


# speedup vs baseline: 8.7793x; 8.7793x over previous
import jax
import jax.numpy as jnp
from jax.experimental import pallas as pl
from jax.experimental.pallas import tpu as pltpu

D_MODEL = 1024
D_STATE = 16
D_CONV = 4
D_INNER = 2048
DT_RANK = 64
LC = 128


def _silu(v):
    return v * jax.nn.sigmoid(v)


def _mamba_body(xn_ref, wi_ref, cw_ref, cb_ref, xp_ref, dtw_ref, dtb_ref,
                wo_ref, al_ref, d_ref, out_ref, h_ref, tail_ref):
    j = pl.program_id(1)

    @pl.when(j == 0)
    def _():
        h_ref[...] = jnp.zeros_like(h_ref)
        tail_ref[...] = jnp.zeros_like(tail_ref)

    xn = xn_ref[0]
    xz = jax.lax.dot_general(xn, wi_ref[...], (((1,), (1,)), ((), ())),
                             preferred_element_type=jnp.float32)
    x_pre = xz[:, :D_INNER]
    z = xz[:, D_INNER:]

    prev = tail_ref[...]
    ext = jnp.concatenate([prev[8 - (D_CONV - 1):], x_pre], axis=0)
    cw = cw_ref[...]
    conv = (ext[0:LC] * cw[0:1]
            + ext[1:LC + 1] * cw[1:2]
            + ext[2:LC + 2] * cw[2:3]
            + ext[3:LC + 3] * cw[3:4]) + cb_ref[...]
    tail_ref[...] = x_pre[LC - 8:]
    x_ssm = _silu(conv)

    xdbl = jax.lax.dot_general(x_ssm.astype(jnp.bfloat16), xp_ref[...],
                               (((1,), (1,)), ((), ())),
                               preferred_element_type=jnp.float32)
    dt_low = xdbl[:, :DT_RANK]
    delta = jax.nn.softplus(
        jax.lax.dot_general(dt_low.astype(jnp.bfloat16), dtw_ref[...],
                            (((1,), (1,)), ((), ())),
                            preferred_element_type=jnp.float32) + dtb_ref[...])

    a_t = -jnp.exp(al_ref[...])
    dx = delta * x_ssm

    y = x_ssm * d_ref[...]
    for n in range(D_STATE):
        a = jnp.exp(delta * a_t[n:n + 1, :])
        u = dx * xdbl[:, DT_RANK + n:DT_RANK + n + 1]
        off = 1
        while off < LC:
            a_sh = jnp.concatenate(
                [jnp.ones((off, D_INNER), jnp.float32), a[:LC - off]], axis=0)
            u_sh = jnp.concatenate(
                [jnp.zeros((off, D_INNER), jnp.float32), u[:LC - off]], axis=0)
            u = u + a * u_sh
            a = a * a_sh
            off *= 2
        hfull = u + a * h_ref[n:n + 1, :]
        y = y + hfull * xdbl[:, DT_RANK + D_STATE + n:DT_RANK + D_STATE + n + 1]
        h_ref[n:n + 1, :] = hfull[LC - 1:LC]

    y = y * _silu(z)
    out_ref[0] = jax.lax.dot_general(y.astype(jnp.bfloat16), wo_ref[...],
                                     (((1,), (1,)), ((), ())),
                                     preferred_element_type=jnp.float32)


def kernel(x_norm, in_proj_w, conv_w, conv_b, x_proj_w, dt_proj_w, dt_proj_b,
           out_proj_w, A_log, D):
    b, l, _ = x_norm.shape
    nc = l // LC
    xn = x_norm.astype(jnp.bfloat16)
    wi = in_proj_w.astype(jnp.bfloat16)
    xp = x_proj_w.astype(jnp.bfloat16)
    dtw = dt_proj_w.astype(jnp.bfloat16)
    wo = out_proj_w.astype(jnp.bfloat16)
    cw = conv_w.T
    cb = conv_b.reshape(1, -1)
    dtb = dt_proj_b.reshape(1, -1)
    al = A_log.T
    dvec = D.reshape(1, -1)

    full = lambda arr: pl.BlockSpec(arr.shape, lambda i, j: (0,) * arr.ndim)
    out = pl.pallas_call(
        _mamba_body,
        out_shape=jax.ShapeDtypeStruct((b, l, D_MODEL), jnp.float32),
        grid=(b, nc),
        in_specs=[
            pl.BlockSpec((1, LC, D_MODEL), lambda i, j: (i, j, 0)),
            full(wi), full(cw), full(cb), full(xp), full(dtw), full(dtb),
            full(wo), full(al), full(dvec),
        ],
        out_specs=pl.BlockSpec((1, LC, D_MODEL), lambda i, j: (i, j, 0)),
        scratch_shapes=[
            pltpu.VMEM((D_STATE, D_INNER), jnp.float32),
            pltpu.VMEM((8, D_INNER), jnp.float32),
        ],
        compiler_params=pltpu.CompilerParams(
            dimension_semantics=("parallel", "arbitrary"),
            vmem_limit_bytes=56 * 1024 * 1024,
        ),
        name="ourmamba_fused",
    )(xn, wi, cw, cb, xp, dtw, dtb, wo, al, dvec)
    return out


# speedup vs baseline: 18.6507x; 2.1244x over previous
import jax
import jax.numpy as jnp
from jax.experimental import pallas as pl
from jax.experimental.pallas import tpu as pltpu

D_MODEL = 1024
D_STATE = 16
D_CONV = 4
D_INNER = 2048
DT_RANK = 64
LC = 128


def _silu(v):
    return v * jax.nn.sigmoid(v)


def _mamba_body(xn_ref, wi_ref, cw_ref, cb_ref, xp_ref, dtw_ref, dtb_ref,
                wo_ref, al_ref, d_ref, out_ref, h_ref, tail_ref, y_scr):
    j = pl.program_id(1)

    @pl.when(j == 0)
    def _():
        h_ref[...] = jnp.zeros_like(h_ref)
        tail_ref[...] = jnp.zeros_like(tail_ref)

    xn = xn_ref[0]
    xz = jax.lax.dot_general(xn, wi_ref[...], (((1,), (1,)), ((), ())),
                             preferred_element_type=jnp.float32)
    x_pre = xz[:, :D_INNER]
    z = xz[:, D_INNER:]

    prev = tail_ref[...]
    ext = jnp.concatenate([prev[8 - (D_CONV - 1):], x_pre], axis=0)
    cw = cw_ref[...]
    conv = (ext[0:LC] * cw[0:1]
            + ext[1:LC + 1] * cw[1:2]
            + ext[2:LC + 2] * cw[2:3]
            + ext[3:LC + 3] * cw[3:4]) + cb_ref[...]
    tail_ref[...] = x_pre[LC - 8:]
    x_ssm = _silu(conv)

    xdbl = jax.lax.dot_general(x_ssm.astype(jnp.bfloat16), xp_ref[...],
                               (((1,), (1,)), ((), ())),
                               preferred_element_type=jnp.float32)
    dt_low = xdbl[:, :DT_RANK]
    delta = jax.nn.softplus(
        jax.lax.dot_general(dt_low.astype(jnp.bfloat16), dtw_ref[...],
                            (((1,), (1,)), ((), ())),
                            preferred_element_type=jnp.float32) + dtb_ref[...])

    a_mat = -jnp.exp(al_ref[...])
    dx = delta * x_ssm

    bct = jnp.swapaxes(xdbl[:, DT_RANK:DT_RANK + 2 * D_STATE], 0, 1)

    h = h_ref[...]
    for t in range(LC):
        drow = delta[t:t + 1, :]
        at = jnp.exp(a_mat * drow)
        u = dx[t:t + 1, :] * bct[0:D_STATE, t:t + 1]
        h = at * h + u
        y_scr[t:t + 1, :] = jnp.sum(
            h * bct[D_STATE:, t:t + 1], axis=0, keepdims=True)
    h_ref[...] = h

    y = (y_scr[...] + x_ssm * d_ref[...]) * _silu(z)
    out_ref[0] = jax.lax.dot_general(y.astype(jnp.bfloat16), wo_ref[...],
                                     (((1,), (1,)), ((), ())),
                                     preferred_element_type=jnp.float32)


def kernel(x_norm, in_proj_w, conv_w, conv_b, x_proj_w, dt_proj_w, dt_proj_b,
           out_proj_w, A_log, D):
    b, l, _ = x_norm.shape
    nc = l // LC
    xn = x_norm.astype(jnp.bfloat16)
    wi = in_proj_w.astype(jnp.bfloat16)
    xp = x_proj_w.astype(jnp.bfloat16)
    dtw = dt_proj_w.astype(jnp.bfloat16)
    wo = out_proj_w.astype(jnp.bfloat16)
    cw = conv_w.T
    cb = conv_b.reshape(1, -1)
    dtb = dt_proj_b.reshape(1, -1)
    al = A_log.T
    dvec = D.reshape(1, -1)

    full = lambda arr: pl.BlockSpec(arr.shape, lambda i, j: (0,) * arr.ndim)
    out = pl.pallas_call(
        _mamba_body,
        out_shape=jax.ShapeDtypeStruct((b, l, D_MODEL), jnp.float32),
        grid=(b, nc),
        in_specs=[
            pl.BlockSpec((1, LC, D_MODEL), lambda i, j: (i, j, 0)),
            full(wi), full(cw), full(cb), full(xp), full(dtw), full(dtb),
            full(wo), full(al), full(dvec),
        ],
        out_specs=pl.BlockSpec((1, LC, D_MODEL), lambda i, j: (i, j, 0)),
        scratch_shapes=[
            pltpu.VMEM((D_STATE, D_INNER), jnp.float32),
            pltpu.VMEM((8, D_INNER), jnp.float32),
            pltpu.VMEM((LC, D_INNER), jnp.float32),
        ],
        compiler_params=pltpu.CompilerParams(
            dimension_semantics=("parallel", "arbitrary"),
            vmem_limit_bytes=56 * 1024 * 1024,
        ),
        name="ourmamba_fused",
    )(xn, wi, cw, cb, xp, dtw, dtb, wo, al, dvec)
    return out
